# baseline (speedup 1.0000x reference)
import functools

import jax
import jax.numpy as jnp
from jax import lax
from jax.experimental import pallas as pl
from jax.experimental.pallas import tpu as pltpu

_EPS = 1e-5


def _h_stats_kernel(x_ref, w1_ref, h_ref, s1_ref):
    xb = x_ref[...].astype(jnp.bfloat16)
    w1b = w1_ref[...].astype(jnp.bfloat16)
    h = jnp.dot(xb, w1b, preferred_element_type=jnp.float32)
    h_ref[...] = h.astype(jnp.bfloat16)
    s1_ref[0, 0:1, :] = jnp.sum(h, axis=0, keepdims=True)
    s1_ref[0, 1:2, :] = jnp.sum(h * h, axis=0, keepdims=True)


def _gram_kernel(h_ref, g1_ref, b1_ref, s1p_ref, gr_ref, sr_ref, *, batch):
    inv_b = 1.0 / batch
    m1 = s1p_ref[0:1, :] * inv_b
    var1 = s1p_ref[1:2, :] * inv_b - m1 * m1
    scale = lax.rsqrt(var1 + _EPS) * g1_ref[...]
    shift = b1_ref[...] - m1 * scale

    h = h_ref[...].astype(jnp.float32)
    r = jnp.maximum(h * scale + shift, 0.0)
    rb = r.astype(jnp.bfloat16)
    gr_ref[...] = lax.dot_general(
        rb, rb, (((0,), (0,)), ((), ())), preferred_element_type=jnp.float32)
    sr_ref[0, 0:1, :] = jnp.sum(r, axis=0, keepdims=True)


def _s1_reduce_kernel(s1p_ref, s1_ref):
    s1_ref[...] = jnp.sum(s1p_ref[...], axis=0)


def _finalize_kernel(grp_ref, srp_ref, w2_ref, s1sum_ref, s1_ref, s2_ref,
                     *, batch, feat, ntiles):
    inv_b = 1.0 / batch
    m1 = s1sum_ref[0:1, :] * inv_b
    var1 = s1sum_ref[1:2, :] * inv_b - m1 * m1
    s1_ref[0:1, :] = m1
    s1_ref[1:2, :] = lax.rsqrt(var1 + _EPS)

    w2 = w2_ref[...]
    g = jnp.sum(grp_ref[...].reshape(ntiles, feat, feat), axis=0)
    sr = jnp.sum(srp_ref[...], axis=0)[0:1, :]
    m2 = jnp.dot(sr, w2, preferred_element_type=jnp.float32,
                 precision=lax.Precision.HIGHEST) * inv_b
    t = jnp.dot(g, w2, preferred_element_type=jnp.float32,
                precision=lax.Precision.HIGHEST)
    e2 = jnp.sum(w2 * t, axis=0, keepdims=True) * inv_b
    var2 = e2 - m2 * m2
    s2_ref[0:1, :] = m2
    s2_ref[1:2, :] = lax.rsqrt(var2 + _EPS)


def _out_kernel(h_ref, g1_ref, b1_ref, w2_ref, s1_ref, s2_ref, o_ref):
    scale = s1_ref[1:2, :] * g1_ref[...]
    shift = b1_ref[...] - s1_ref[0:1, :] * scale
    h = h_ref[...].astype(jnp.float32)
    r = jnp.maximum(h * scale + shift, 0.0)
    w2b = w2_ref[...].astype(jnp.bfloat16)
    y = jnp.dot(r.astype(jnp.bfloat16), w2b,
                preferred_element_type=jnp.float32)
    o_ref[...] = ((y - s2_ref[0:1, :]) * s2_ref[1:2, :]).astype(o_ref.dtype)


def kernel(x, w1, g1, b1, w2):
    B, D = x.shape
    P = w2.shape[1]

    tb = None
    for cand in (8192, 4096, 2048, 1024, 512, 256, 128, 8):
        if B % (2 * cand) == 0:
            tb = cand
            break
    nb = B // tb

    h, s1p = pl.pallas_call(
        _h_stats_kernel,
        out_shape=(jax.ShapeDtypeStruct((B, D), jnp.bfloat16),
                   jax.ShapeDtypeStruct((nb, 8, D), jnp.float32)),
        grid=(nb,),
        in_specs=[
            pl.BlockSpec((tb, D), lambda b: (b, 0)),
            pl.BlockSpec((D, D), lambda b: (0, 0)),
        ],
        out_specs=(
            pl.BlockSpec((tb, D), lambda b: (b, 0)),
            pl.BlockSpec((1, 8, D), lambda b: (b, 0, 0)),
        ),
        compiler_params=pltpu.CompilerParams(
            dimension_semantics=("parallel",)),
    )(x, w1)

    s1sum = pl.pallas_call(
        _s1_reduce_kernel,
        out_shape=jax.ShapeDtypeStruct((8, D), jnp.float32),
    )(s1p)

    gram_kernel = functools.partial(_gram_kernel, batch=B)
    grp, srp = pl.pallas_call(
        gram_kernel,
        out_shape=(jax.ShapeDtypeStruct((nb * D, D), jnp.float32),
                   jax.ShapeDtypeStruct((nb, 8, D), jnp.float32)),
        grid=(nb,),
        in_specs=[
            pl.BlockSpec((tb, D), lambda b: (b, 0)),
            pl.BlockSpec((1, D), lambda b: (0, 0)),
            pl.BlockSpec((1, D), lambda b: (0, 0)),
            pl.BlockSpec((8, D), lambda b: (0, 0)),
        ],
        out_specs=(
            pl.BlockSpec((D, D), lambda b: (b, 0)),
            pl.BlockSpec((1, 8, D), lambda b: (b, 0, 0)),
        ),
        compiler_params=pltpu.CompilerParams(
            dimension_semantics=("parallel",)),
    )(h, g1, b1, s1sum)

    finalize_kernel = functools.partial(
        _finalize_kernel, batch=B, feat=D, ntiles=nb)
    s1, s2 = pl.pallas_call(
        finalize_kernel,
        out_shape=(jax.ShapeDtypeStruct((8, D), jnp.float32),
                   jax.ShapeDtypeStruct((8, P), jnp.float32)),
    )(grp, srp, w2, s1sum)

    out = pl.pallas_call(
        _out_kernel,
        out_shape=jax.ShapeDtypeStruct((B, P), x.dtype),
        grid=(nb,),
        in_specs=[
            pl.BlockSpec((tb, D), lambda b: (b, 0)),
            pl.BlockSpec((1, D), lambda b: (0, 0)),
            pl.BlockSpec((1, D), lambda b: (0, 0)),
            pl.BlockSpec((D, P), lambda b: (0, 0)),
            pl.BlockSpec((8, D), lambda b: (0, 0)),
            pl.BlockSpec((8, P), lambda b: (0, 0)),
        ],
        out_specs=pl.BlockSpec((tb, P), lambda b: (b, 0)),
        compiler_params=pltpu.CompilerParams(
            dimension_semantics=("parallel",)),
    )(h, g1, b1, w2, s1, s2)
    return out


# speedup vs baseline: 1.0428x; 1.0428x over previous
import functools

import jax
import jax.numpy as jnp
from jax import lax
from jax.experimental import pallas as pl
from jax.experimental.pallas import tpu as pltpu

_EPS = 1e-5


def _h_stats_kernel(x_ref, w1_ref, h_ref, s1_ref):
    xb = x_ref[...].astype(jnp.bfloat16)
    w1b = w1_ref[...].astype(jnp.bfloat16)
    h = jnp.dot(xb, w1b, preferred_element_type=jnp.float32)
    h_ref[...] = h.astype(jnp.bfloat16)
    s1_ref[0, 0:1, :] = jnp.sum(h, axis=0, keepdims=True)
    s1_ref[0, 1:2, :] = jnp.sum(h * h, axis=0, keepdims=True)


def _gram_kernel(ones_ref, h_ref, g1_ref, b1_ref, s1p_ref, gr_ref, sr_ref,
                 *, batch):
    inv_b = 1.0 / batch
    s1sum = jnp.sum(s1p_ref[...], axis=0)
    m1 = s1sum[0:1, :] * inv_b
    var1 = s1sum[1:2, :] * inv_b - m1 * m1
    scale = lax.rsqrt(var1 + _EPS) * g1_ref[...]
    shift = b1_ref[...] - m1 * scale

    h = h_ref[...]
    rb = jnp.maximum(h * scale.astype(jnp.bfloat16)
                     + shift.astype(jnp.bfloat16), 0.0)
    gr_ref[...] = lax.dot_general(
        rb, rb, (((0,), (0,)), ((), ())), preferred_element_type=jnp.float32)
    sr_ref[0] = jnp.dot(ones_ref[...], rb,
                        preferred_element_type=jnp.float32)


def _finalize_kernel(s1p_ref, grp_ref, srp_ref, w2_ref, s1_ref, s2_ref,
                     *, batch, feat, ntiles):
    inv_b = 1.0 / batch
    s1sum = jnp.sum(s1p_ref[...], axis=0)
    m1 = s1sum[0:1, :] * inv_b
    var1 = s1sum[1:2, :] * inv_b - m1 * m1
    s1_ref[0:1, :] = m1
    s1_ref[1:2, :] = lax.rsqrt(var1 + _EPS)

    w2 = w2_ref[...]
    g = jnp.sum(grp_ref[...].reshape(ntiles, feat, feat), axis=0)
    sr = jnp.sum(srp_ref[...], axis=0)[0:1, :]
    m2 = jnp.dot(sr, w2, preferred_element_type=jnp.float32,
                 precision=lax.Precision.HIGHEST) * inv_b
    t = jnp.dot(g, w2, preferred_element_type=jnp.float32,
                precision=lax.Precision.HIGHEST)
    e2 = jnp.sum(w2 * t, axis=0, keepdims=True) * inv_b
    var2 = e2 - m2 * m2
    s2_ref[0:1, :] = m2
    s2_ref[1:2, :] = lax.rsqrt(var2 + _EPS)


def _out_kernel(h_ref, g1_ref, b1_ref, w2_ref, s1_ref, s2_ref, o_ref):
    scale = s1_ref[1:2, :] * g1_ref[...]
    shift = b1_ref[...] - s1_ref[0:1, :] * scale
    h = h_ref[...].astype(jnp.float32)
    r = jnp.maximum(h * scale + shift, 0.0)
    w2b = w2_ref[...].astype(jnp.bfloat16)
    y = jnp.dot(r.astype(jnp.bfloat16), w2b,
                preferred_element_type=jnp.float32)
    o_ref[...] = ((y - s2_ref[0:1, :]) * s2_ref[1:2, :]).astype(o_ref.dtype)


def kernel(x, w1, g1, b1, w2):
    B, D = x.shape
    P = w2.shape[1]

    tb = None
    for cand in (8192, 4096, 2048, 1024, 512, 256, 128, 8):
        if B % (2 * cand) == 0:
            tb = cand
            break
    nb = B // tb

    h, s1p = pl.pallas_call(
        _h_stats_kernel,
        out_shape=(jax.ShapeDtypeStruct((B, D), jnp.bfloat16),
                   jax.ShapeDtypeStruct((nb, 8, D), jnp.float32)),
        grid=(nb,),
        in_specs=[
            pl.BlockSpec((tb, D), lambda b: (b, 0)),
            pl.BlockSpec((D, D), lambda b: (0, 0)),
        ],
        out_specs=(
            pl.BlockSpec((tb, D), lambda b: (b, 0)),
            pl.BlockSpec((1, 8, D), lambda b: (b, 0, 0)),
        ),
        compiler_params=pltpu.CompilerParams(
            dimension_semantics=("arbitrary",)),
    )(x, w1)

    ones = jnp.ones((8, tb), jnp.bfloat16)
    gram_kernel = functools.partial(_gram_kernel, batch=B)
    grp, srp = pl.pallas_call(
        gram_kernel,
        out_shape=(jax.ShapeDtypeStruct((nb * D, D), jnp.float32),
                   jax.ShapeDtypeStruct((nb, 8, D), jnp.float32)),
        grid=(nb,),
        in_specs=[
            pl.BlockSpec((8, tb), lambda b: (0, 0)),
            pl.BlockSpec((tb, D), lambda b: (b, 0)),
            pl.BlockSpec((1, D), lambda b: (0, 0)),
            pl.BlockSpec((1, D), lambda b: (0, 0)),
            pl.BlockSpec((nb, 8, D), lambda b: (0, 0, 0)),
        ],
        out_specs=(
            pl.BlockSpec((D, D), lambda b: (b, 0)),
            pl.BlockSpec((1, 8, D), lambda b: (b, 0, 0)),
        ),
        compiler_params=pltpu.CompilerParams(
            dimension_semantics=("arbitrary",)),
    )(ones, h, g1, b1, s1p)

    finalize_kernel = functools.partial(
        _finalize_kernel, batch=B, feat=D, ntiles=nb)
    s1, s2 = pl.pallas_call(
        finalize_kernel,
        out_shape=(jax.ShapeDtypeStruct((8, D), jnp.float32),
                   jax.ShapeDtypeStruct((8, P), jnp.float32)),
    )(s1p, grp, srp, w2)

    out = pl.pallas_call(
        _out_kernel,
        out_shape=jax.ShapeDtypeStruct((B, P), x.dtype),
        grid=(nb,),
        in_specs=[
            pl.BlockSpec((tb, D), lambda b: (b, 0)),
            pl.BlockSpec((1, D), lambda b: (0, 0)),
            pl.BlockSpec((1, D), lambda b: (0, 0)),
            pl.BlockSpec((D, P), lambda b: (0, 0)),
            pl.BlockSpec((8, D), lambda b: (0, 0)),
            pl.BlockSpec((8, P), lambda b: (0, 0)),
        ],
        out_specs=pl.BlockSpec((tb, P), lambda b: (b, 0)),
        compiler_params=pltpu.CompilerParams(
            dimension_semantics=("arbitrary",)),
    )(h, g1, b1, w2, s1, s2)
    return out


# speedup vs baseline: 1.0682x; 1.0243x over previous
import functools

import jax
import jax.numpy as jnp
from jax import lax
from jax.experimental import pallas as pl
from jax.experimental.pallas import tpu as pltpu

_EPS = 1e-5


def _h_stats_kernel(x_ref, w1_ref, h_ref, s1_ref):
    xb = x_ref[...].astype(jnp.bfloat16)
    w1b = w1_ref[...].astype(jnp.bfloat16)
    h = jnp.dot(xb, w1b, preferred_element_type=jnp.float32)
    h_ref[...] = h.astype(jnp.bfloat16)
    s1_ref[0, 0:1, :] = jnp.sum(h, axis=0, keepdims=True)
    s1_ref[0, 1:2, :] = jnp.sum(h * h, axis=0, keepdims=True)


def _gram_kernel(h_ref, g1_ref, b1_ref, s1p_ref, w2_ref, s1_ref, s2_ref,
                 gr_ref, sr_ref, *, batch, tile_b):
    b = pl.program_id(0)
    nb = pl.num_programs(0)

    @pl.when(b == 0)
    def _init():
        gr_ref[...] = jnp.zeros_like(gr_ref)
        sr_ref[...] = jnp.zeros_like(sr_ref)

    inv_b = 1.0 / batch
    s1sum = jnp.sum(s1p_ref[...], axis=0)
    m1 = s1sum[0:1, :] * inv_b
    var1 = s1sum[1:2, :] * inv_b - m1 * m1
    rstd1 = lax.rsqrt(var1 + _EPS)
    scale = rstd1 * g1_ref[...]
    shift = b1_ref[...] - m1 * scale

    h = h_ref[...]
    rb = jnp.maximum(h * scale.astype(jnp.bfloat16)
                     + shift.astype(jnp.bfloat16), 0.0)
    gr_ref[...] += lax.dot_general(
        rb, rb, (((0,), (0,)), ((), ())), preferred_element_type=jnp.float32)
    ones = jnp.ones((8, tile_b), jnp.bfloat16)
    sr_ref[...] += jnp.dot(ones, rb, preferred_element_type=jnp.float32)

    @pl.when(b == nb - 1)
    def _finalize():
        s1_ref[0:1, :] = m1
        s1_ref[1:2, :] = rstd1
        w2 = w2_ref[...]
        sr = sr_ref[0:1, :]
        m2 = jnp.dot(sr, w2, preferred_element_type=jnp.float32,
                     precision=lax.Precision.HIGHEST) * inv_b
        t = jnp.dot(gr_ref[...], w2, preferred_element_type=jnp.float32,
                    precision=lax.Precision.HIGHEST)
        e2 = jnp.sum(w2 * t, axis=0, keepdims=True) * inv_b
        var2 = e2 - m2 * m2
        s2_ref[0:1, :] = m2
        s2_ref[1:2, :] = lax.rsqrt(var2 + _EPS)


def _out_kernel(h_ref, g1_ref, b1_ref, w2_ref, s1_ref, s2_ref, o_ref):
    scale = s1_ref[1:2, :] * g1_ref[...]
    shift = b1_ref[...] - s1_ref[0:1, :] * scale
    h = h_ref[...].astype(jnp.float32)
    r = jnp.maximum(h * scale + shift, 0.0)
    w2b = w2_ref[...].astype(jnp.bfloat16)
    y = jnp.dot(r.astype(jnp.bfloat16), w2b,
                preferred_element_type=jnp.float32)
    o_ref[...] = ((y - s2_ref[0:1, :]) * s2_ref[1:2, :]).astype(o_ref.dtype)


def kernel(x, w1, g1, b1, w2):
    B, D = x.shape
    P = w2.shape[1]

    tb = None
    for cand in (8192, 4096, 2048, 1024, 512, 256, 128, 8):
        if B % (2 * cand) == 0:
            tb = cand
            break
    nb = B // tb

    h, s1p = pl.pallas_call(
        _h_stats_kernel,
        out_shape=(jax.ShapeDtypeStruct((B, D), jnp.bfloat16),
                   jax.ShapeDtypeStruct((nb, 8, D), jnp.float32)),
        grid=(nb,),
        in_specs=[
            pl.BlockSpec((tb, D), lambda b: (b, 0)),
            pl.BlockSpec((D, D), lambda b: (0, 0)),
        ],
        out_specs=(
            pl.BlockSpec((tb, D), lambda b: (b, 0)),
            pl.BlockSpec((1, 8, D), lambda b: (b, 0, 0)),
        ),
        compiler_params=pltpu.CompilerParams(
            dimension_semantics=("arbitrary",)),
    )(x, w1)

    gram_kernel = functools.partial(_gram_kernel, batch=B, tile_b=tb)
    s1, s2 = pl.pallas_call(
        gram_kernel,
        out_shape=(jax.ShapeDtypeStruct((8, D), jnp.float32),
                   jax.ShapeDtypeStruct((8, P), jnp.float32)),
        grid=(nb,),
        in_specs=[
            pl.BlockSpec((tb, D), lambda b: (b, 0)),
            pl.BlockSpec((1, D), lambda b: (0, 0)),
            pl.BlockSpec((1, D), lambda b: (0, 0)),
            pl.BlockSpec((nb, 8, D), lambda b: (0, 0, 0)),
            pl.BlockSpec((D, P), lambda b: (0, 0)),
        ],
        out_specs=(
            pl.BlockSpec((8, D), lambda b: (0, 0)),
            pl.BlockSpec((8, P), lambda b: (0, 0)),
        ),
        scratch_shapes=[
            pltpu.VMEM((D, D), jnp.float32),
            pltpu.VMEM((8, D), jnp.float32),
        ],
        compiler_params=pltpu.CompilerParams(
            dimension_semantics=("arbitrary",)),
    )(h, g1, b1, s1p, w2)

    out = pl.pallas_call(
        _out_kernel,
        out_shape=jax.ShapeDtypeStruct((B, P), x.dtype),
        grid=(nb,),
        in_specs=[
            pl.BlockSpec((tb, D), lambda b: (b, 0)),
            pl.BlockSpec((1, D), lambda b: (0, 0)),
            pl.BlockSpec((1, D), lambda b: (0, 0)),
            pl.BlockSpec((D, P), lambda b: (0, 0)),
            pl.BlockSpec((8, D), lambda b: (0, 0)),
            pl.BlockSpec((8, P), lambda b: (0, 0)),
        ],
        out_specs=pl.BlockSpec((tb, P), lambda b: (b, 0)),
        compiler_params=pltpu.CompilerParams(
            dimension_semantics=("arbitrary",)),
    )(h, g1, b1, w2, s1, s2)
    return out


# speedup vs baseline: 1.0977x; 1.0276x over previous
import functools

import jax
import jax.numpy as jnp
from jax import lax
from jax.experimental import pallas as pl
from jax.experimental.pallas import tpu as pltpu

_EPS = 1e-5


def _h_stats_kernel(x_ref, w1_ref, h_ref, s1_ref):
    xb = x_ref[...].astype(jnp.bfloat16)
    w1b = w1_ref[...].astype(jnp.bfloat16)
    h = jnp.dot(xb, w1b, preferred_element_type=jnp.float32)
    h_ref[...] = h.astype(jnp.bfloat16)
    s1_ref[0, 0:1, :] = jnp.sum(h, axis=0, keepdims=True)
    s1_ref[0, 1:2, :] = jnp.sum(h * h, axis=0, keepdims=True)


def _gram_kernel(h_ref, g1_ref, b1_ref, s1p_ref, w2_ref, s1_ref, s2_ref,
                 gr_ref, sr_ref, *, batch, tile_b):
    b = pl.program_id(0)
    nb = pl.num_programs(0)

    @pl.when(b == 0)
    def _init():
        gr_ref[...] = jnp.zeros_like(gr_ref)
        sr_ref[...] = jnp.zeros_like(sr_ref)

    inv_b = 1.0 / batch
    s1sum = jnp.sum(s1p_ref[...], axis=0)
    m1 = s1sum[0:1, :] * inv_b
    var1 = s1sum[1:2, :] * inv_b - m1 * m1
    rstd1 = lax.rsqrt(var1 + _EPS)
    scale = rstd1 * g1_ref[...]
    shift = b1_ref[...] - m1 * scale

    h = h_ref[...]
    rb = jnp.maximum(h * scale.astype(jnp.bfloat16)
                     + shift.astype(jnp.bfloat16), 0.0)
    gr_ref[...] += lax.dot_general(
        rb, rb, (((0,), (0,)), ((), ())), preferred_element_type=jnp.float32)
    ones = jnp.ones((8, tile_b), jnp.bfloat16)
    sr_ref[...] += jnp.dot(ones, rb, preferred_element_type=jnp.float32)

    @pl.when(b == nb - 1)
    def _finalize():
        s1_ref[0:1, :] = m1
        s1_ref[1:2, :] = rstd1
        w2 = w2_ref[...]
        sr = sr_ref[0:1, :]
        m2 = jnp.dot(sr, w2, preferred_element_type=jnp.float32,
                     precision=lax.Precision.HIGHEST) * inv_b
        t = jnp.dot(gr_ref[...], w2, preferred_element_type=jnp.float32,
                    precision=lax.Precision.HIGHEST)
        e2 = jnp.sum(w2 * t, axis=0, keepdims=True) * inv_b
        var2 = e2 - m2 * m2
        s2_ref[0:1, :] = m2
        s2_ref[1:2, :] = lax.rsqrt(var2 + _EPS)


def _out_kernel(h_ref, g1_ref, b1_ref, w2_ref, s1_ref, s2_ref, o_ref):
    scale = s1_ref[1:2, :] * g1_ref[...]
    shift = b1_ref[...] - s1_ref[0:1, :] * scale
    h = h_ref[...].astype(jnp.float32)
    r = jnp.maximum(h * scale + shift, 0.0)
    w2b = w2_ref[...].astype(jnp.bfloat16)
    y = jnp.dot(r.astype(jnp.bfloat16), w2b,
                preferred_element_type=jnp.float32)
    o_ref[...] = ((y - s2_ref[0:1, :]) * s2_ref[1:2, :]).astype(o_ref.dtype)


def kernel(x, w1, g1, b1, w2):
    B, D = x.shape
    P = w2.shape[1]

    def _pick_tile(limit):
        for cand in (limit, 8192, 4096, 2048, 1024, 512, 256, 128, 8):
            if cand <= limit and B % (2 * cand) == 0:
                return cand
        return B

    tb1 = _pick_tile(16384)
    tb = _pick_tile(8192)
    nb1 = B // tb1
    nb = B // tb

    h, s1p = pl.pallas_call(
        _h_stats_kernel,
        out_shape=(jax.ShapeDtypeStruct((B, D), jnp.bfloat16),
                   jax.ShapeDtypeStruct((nb1, 8, D), jnp.float32)),
        grid=(nb1,),
        in_specs=[
            pl.BlockSpec((tb1, D), lambda b: (b, 0)),
            pl.BlockSpec((D, D), lambda b: (0, 0)),
        ],
        out_specs=(
            pl.BlockSpec((tb1, D), lambda b: (b, 0)),
            pl.BlockSpec((1, 8, D), lambda b: (b, 0, 0)),
        ),
        compiler_params=pltpu.CompilerParams(
            dimension_semantics=("arbitrary",)),
    )(x, w1)

    gram_kernel = functools.partial(_gram_kernel, batch=B, tile_b=tb1)
    s1, s2 = pl.pallas_call(
        gram_kernel,
        out_shape=(jax.ShapeDtypeStruct((8, D), jnp.float32),
                   jax.ShapeDtypeStruct((8, P), jnp.float32)),
        grid=(nb1,),
        in_specs=[
            pl.BlockSpec((tb1, D), lambda b: (b, 0)),
            pl.BlockSpec((1, D), lambda b: (0, 0)),
            pl.BlockSpec((1, D), lambda b: (0, 0)),
            pl.BlockSpec((nb1, 8, D), lambda b: (0, 0, 0)),
            pl.BlockSpec((D, P), lambda b: (0, 0)),
        ],
        out_specs=(
            pl.BlockSpec((8, D), lambda b: (0, 0)),
            pl.BlockSpec((8, P), lambda b: (0, 0)),
        ),
        scratch_shapes=[
            pltpu.VMEM((D, D), jnp.float32),
            pltpu.VMEM((8, D), jnp.float32),
        ],
        compiler_params=pltpu.CompilerParams(
            dimension_semantics=("arbitrary",)),
    )(h, g1, b1, s1p, w2)

    out = pl.pallas_call(
        _out_kernel,
        out_shape=jax.ShapeDtypeStruct((B, P), x.dtype),
        grid=(nb,),
        in_specs=[
            pl.BlockSpec((tb, D), lambda b: (b, 0)),
            pl.BlockSpec((1, D), lambda b: (0, 0)),
            pl.BlockSpec((1, D), lambda b: (0, 0)),
            pl.BlockSpec((D, P), lambda b: (0, 0)),
            pl.BlockSpec((8, D), lambda b: (0, 0)),
            pl.BlockSpec((8, P), lambda b: (0, 0)),
        ],
        out_specs=pl.BlockSpec((tb, P), lambda b: (b, 0)),
        compiler_params=pltpu.CompilerParams(
            dimension_semantics=("arbitrary",)),
    )(h, g1, b1, w2, s1, s2)
    return out


# speedup vs baseline: 1.1208x; 1.0210x over previous
import functools

import jax
import jax.numpy as jnp
from jax import lax
from jax.experimental import pallas as pl
from jax.experimental.pallas import tpu as pltpu

_EPS = 1e-5


def _h_stats_kernel(x_ref, w1_ref, h_ref, s1_ref):
    xb = x_ref[...].astype(jnp.bfloat16)
    w1b = w1_ref[...].astype(jnp.bfloat16)
    h = jnp.dot(xb, w1b, preferred_element_type=jnp.float32)
    h_ref[...] = h.astype(jnp.bfloat16)
    s1_ref[0, 0:1, :] = jnp.sum(h, axis=0, keepdims=True)
    s1_ref[0, 1:2, :] = jnp.sum(h * h, axis=0, keepdims=True)


def _gram_kernel(h_ref, g1_ref, b1_ref, s1p_ref, w2_ref, s1_ref, s2_ref,
                 gr_ref, sr_ref, *, batch, tile_b):
    b = pl.program_id(0)
    nb = pl.num_programs(0)

    @pl.when(b == 0)
    def _init():
        gr_ref[...] = jnp.zeros_like(gr_ref)
        sr_ref[...] = jnp.zeros_like(sr_ref)

    inv_b = 1.0 / batch
    s1sum = jnp.sum(s1p_ref[...], axis=0)
    m1 = s1sum[0:1, :] * inv_b
    var1 = s1sum[1:2, :] * inv_b - m1 * m1
    rstd1 = lax.rsqrt(var1 + _EPS)
    scale = rstd1 * g1_ref[...]
    shift = b1_ref[...] - m1 * scale

    h = h_ref[...]
    rb = jnp.maximum(h * scale.astype(jnp.bfloat16)
                     + shift.astype(jnp.bfloat16), 0.0)
    gr_ref[...] += lax.dot_general(
        rb, rb, (((0,), (0,)), ((), ())), preferred_element_type=jnp.float32)
    tile_sum = jnp.sum(rb.reshape(tile_b // 256, 256, -1), axis=0)
    sr_ref[0:1, :] += jnp.sum(tile_sum.astype(jnp.float32), axis=0,
                              keepdims=True)

    @pl.when(b == nb - 1)
    def _finalize():
        s1_ref[0:1, :] = m1
        s1_ref[1:2, :] = rstd1
        w2 = w2_ref[...]
        sr = sr_ref[0:1, :]
        m2 = jnp.dot(sr, w2, preferred_element_type=jnp.float32,
                     precision=lax.Precision.HIGHEST) * inv_b
        t = jnp.dot(gr_ref[...], w2, preferred_element_type=jnp.float32,
                    precision=lax.Precision.HIGHEST)
        e2 = jnp.sum(w2 * t, axis=0, keepdims=True) * inv_b
        var2 = e2 - m2 * m2
        s2_ref[0:1, :] = m2
        s2_ref[1:2, :] = lax.rsqrt(var2 + _EPS)


def _out_kernel(h_ref, g1_ref, b1_ref, w2_ref, s1_ref, s2_ref, o_ref):
    scale = s1_ref[1:2, :] * g1_ref[...]
    shift = b1_ref[...] - s1_ref[0:1, :] * scale
    h = h_ref[...].astype(jnp.float32)
    r = jnp.maximum(h * scale + shift, 0.0)
    w2b = w2_ref[...].astype(jnp.bfloat16)
    y = jnp.dot(r.astype(jnp.bfloat16), w2b,
                preferred_element_type=jnp.float32)
    o_ref[...] = ((y - s2_ref[0:1, :]) * s2_ref[1:2, :]).astype(o_ref.dtype)


def kernel(x, w1, g1, b1, w2):
    B, D = x.shape
    P = w2.shape[1]

    def _pick_tile(limit):
        for cand in (limit, 8192, 4096, 2048, 1024, 512, 256, 128, 8):
            if cand <= limit and B % (2 * cand) == 0:
                return cand
        return B

    tb1 = _pick_tile(16384)
    tb = _pick_tile(8192)
    nb1 = B // tb1
    nb = B // tb

    h, s1p = pl.pallas_call(
        _h_stats_kernel,
        out_shape=(jax.ShapeDtypeStruct((B, D), jnp.bfloat16),
                   jax.ShapeDtypeStruct((nb1, 8, D), jnp.float32)),
        grid=(nb1,),
        in_specs=[
            pl.BlockSpec((tb1, D), lambda b: (b, 0)),
            pl.BlockSpec((D, D), lambda b: (0, 0)),
        ],
        out_specs=(
            pl.BlockSpec((tb1, D), lambda b: (b, 0)),
            pl.BlockSpec((1, 8, D), lambda b: (b, 0, 0)),
        ),
        compiler_params=pltpu.CompilerParams(
            dimension_semantics=("arbitrary",)),
    )(x, w1)

    gram_kernel = functools.partial(_gram_kernel, batch=B, tile_b=tb1)
    s1, s2 = pl.pallas_call(
        gram_kernel,
        out_shape=(jax.ShapeDtypeStruct((8, D), jnp.float32),
                   jax.ShapeDtypeStruct((8, P), jnp.float32)),
        grid=(nb1,),
        in_specs=[
            pl.BlockSpec((tb1, D), lambda b: (b, 0)),
            pl.BlockSpec((1, D), lambda b: (0, 0)),
            pl.BlockSpec((1, D), lambda b: (0, 0)),
            pl.BlockSpec((nb1, 8, D), lambda b: (0, 0, 0)),
            pl.BlockSpec((D, P), lambda b: (0, 0)),
        ],
        out_specs=(
            pl.BlockSpec((8, D), lambda b: (0, 0)),
            pl.BlockSpec((8, P), lambda b: (0, 0)),
        ),
        scratch_shapes=[
            pltpu.VMEM((D, D), jnp.float32),
            pltpu.VMEM((8, D), jnp.float32),
        ],
        compiler_params=pltpu.CompilerParams(
            dimension_semantics=("arbitrary",)),
    )(h, g1, b1, s1p, w2)

    out = pl.pallas_call(
        _out_kernel,
        out_shape=jax.ShapeDtypeStruct((B, P), x.dtype),
        grid=(nb,),
        in_specs=[
            pl.BlockSpec((tb, D), lambda b: (b, 0)),
            pl.BlockSpec((1, D), lambda b: (0, 0)),
            pl.BlockSpec((1, D), lambda b: (0, 0)),
            pl.BlockSpec((D, P), lambda b: (0, 0)),
            pl.BlockSpec((8, D), lambda b: (0, 0)),
            pl.BlockSpec((8, P), lambda b: (0, 0)),
        ],
        out_specs=pl.BlockSpec((tb, P), lambda b: (b, 0)),
        compiler_params=pltpu.CompilerParams(
            dimension_semantics=("arbitrary",)),
    )(h, g1, b1, w2, s1, s2)
    return out
